# baseline (speedup 1.0000x reference)
import functools

import jax
import jax.numpy as jnp
from jax import lax
from jax.experimental import pallas as pl
from jax.experimental.pallas import tpu as pltpu
from jax.experimental.pallas import tpu_sc as plsc


def _lookup_kernel(B, D, V, NW, b_per_w, C):
    mesh = plsc.VectorSubcoreMesh(core_axis_name="c", subcore_axis_name="s")
    n_chunks = b_per_w // C
    n_pieces = D // 16
    GR = 16

    @functools.partial(
        pl.kernel,
        mesh=mesh,
        out_type=jax.ShapeDtypeStruct((B, D), jnp.float32),
        scratch_types=[
            pltpu.VMEM((b_per_w,), jnp.int32),
            pltpu.VMEM((V, D), jnp.float32),
            pltpu.VMEM((2, C, D), jnp.float32),
            pltpu.SemaphoreType.DMA,
            pltpu.SemaphoreType.DMA,
        ],
    )
    def k(x_hbm, table_hbm, out_hbm, idx_v, table_v, rows_v, s0, s1):
        wid = lax.axis_index("s") * 2 + lax.axis_index("c")
        base = wid * b_per_w
        pltpu.sync_copy(x_hbm.at[pl.ds(base, b_per_w)], idx_v)
        pltpu.sync_copy(table_hbm, table_v)

        ssems = (s0, s1)

        def store(c, b):
            return pltpu.make_async_copy(
                rows_v.at[b], out_hbm.at[pl.ds(base + c * C, C)], ssems[b]
            )

        def assemble(c, b):
            def grp(g, _):
                row0 = g * GR
                iv = idx_v[pl.ds(c * C + row0, GR)]
                for r in range(GR):
                    xi = iv[r]
                    for p in range(n_pieces):
                        rows_v[b, row0 + r, pl.ds(p * 16, 16)] = table_v[
                            xi, pl.ds(p * 16, 16)
                        ]
                return _

            lax.fori_loop(0, C // GR, grp, None)

        assemble(0, 0)
        store(0, 0).start()

        def body(c, _):
            for b in range(2):
                cc = c + b
                nxt = 1 - b
                @pl.when(cc >= 1)
                def _():
                    store(cc - 1, nxt).wait()

                @pl.when(cc + 1 < n_chunks)
                def _():
                    assemble(cc + 1, nxt)
                    store(cc + 1, nxt).start()

            return _

        lax.fori_loop(0, n_chunks // 2, lambda c, u: body(c * 2, u), None)
        store(n_chunks - 1, (n_chunks - 1) % 2).wait()

    return k


def kernel(x, table):
    S, J = x.shape
    V, D = table.shape
    B = S * J
    NW = 32
    b_per_w = B // NW
    C = 64
    xf = x.reshape(B).astype(jnp.int32)
    out = _lookup_kernel(B, D, V, NW, b_per_w, C)(xf, table)
    return out.reshape(S, J, D)
